# baseline (speedup 1.0000x reference)
import jax
import jax.numpy as jnp
from jax.experimental import pallas as pl
from jax.experimental.pallas import tpu as pltpu

_B = 4096
_S = 200
_H = 64
_BS = 4


def _plm_kernel(ids_ref, emb_ref, x_ref, o_ref, pos_ref):
    step = pl.program_id(0)

    @pl.when(step == 0)
    def _():
        ids = ids_ref[...]
        nz = (ids != 0).astype(jnp.int32)
        cnt = jnp.sum(nz, axis=0, keepdims=True)
        pos = jnp.clip(cnt - 1, 0, _S - 1)
        s_iota = jax.lax.broadcasted_iota(jnp.int32, ids.shape, 0)
        idv = jnp.sum(jnp.where(s_iota == pos, ids, 0), axis=0, keepdims=True)
        pos_ref[...] = jnp.where(idv != 0, pos, -1)

    pos = pos_ref[...]
    x = x_ref[...]
    srel = (pos - step * _BS).reshape(1, 1, _B)
    sl_iota = jax.lax.broadcasted_iota(jnp.int32, (_BS, 1, _B), 0)
    m = sl_iota == srel
    o_ref[...] = jnp.where(m, emb_ref[...], x)


def kernel(inputs, item_ids, masked_item_embedding):
    x_t = jnp.transpose(inputs, (1, 2, 0))
    ids_t = jnp.transpose(item_ids, (1, 0))
    emb3 = masked_item_embedding.reshape(1, _H, 1)
    out_t = pl.pallas_call(
        _plm_kernel,
        grid=(_S // _BS,),
        in_specs=[
            pl.BlockSpec((_S, _B), lambda i: (0, 0)),
            pl.BlockSpec((1, _H, 1), lambda i: (0, 0, 0)),
            pl.BlockSpec((_BS, _H, _B), lambda i: (i, 0, 0)),
        ],
        out_specs=pl.BlockSpec((_BS, _H, _B), lambda i: (i, 0, 0)),
        out_shape=jax.ShapeDtypeStruct((_S, _H, _B), inputs.dtype),
        scratch_shapes=[pltpu.VMEM((1, _B), jnp.int32)],
    )(ids_t, emb3, x_t)
    return jnp.transpose(out_t, (2, 0, 1))


# speedup vs baseline: 1.0233x; 1.0233x over previous
import jax
import jax.numpy as jnp
from jax.experimental import pallas as pl
from jax.experimental.pallas import tpu as pltpu

_B = 4096
_S = 200
_H = 64
_BS = 10


def _plm_kernel(ids_ref, emb_ref, x_ref, o_ref, pos_ref):
    step = pl.program_id(0)

    @pl.when(step == 0)
    def _():
        ids = ids_ref[...]
        nz = (ids != 0).astype(jnp.int32)
        cnt = jnp.sum(nz, axis=0, keepdims=True)
        pos = jnp.clip(cnt - 1, 0, _S - 1)
        s_iota = jax.lax.broadcasted_iota(jnp.int32, ids.shape, 0)
        idv = jnp.sum(jnp.where(s_iota == pos, ids, 0), axis=0, keepdims=True)
        pos_ref[...] = jnp.where(idv != 0, pos, -1)

    pos = pos_ref[...]
    x = x_ref[...]
    srel = (pos - step * _BS).reshape(1, 1, _B)
    sl_iota = jax.lax.broadcasted_iota(jnp.int32, (_BS, 1, _B), 0)
    m = sl_iota == srel
    o_ref[...] = jnp.where(m, emb_ref[...], x)


def kernel(inputs, item_ids, masked_item_embedding):
    x_t = jnp.transpose(inputs, (1, 2, 0))
    ids_t = jnp.transpose(item_ids, (1, 0))
    emb3 = masked_item_embedding.reshape(1, _H, 1)
    out_t = pl.pallas_call(
        _plm_kernel,
        grid=(_S // _BS,),
        in_specs=[
            pl.BlockSpec((_S, _B), lambda i: (0, 0)),
            pl.BlockSpec((1, _H, 1), lambda i: (0, 0, 0)),
            pl.BlockSpec((_BS, _H, _B), lambda i: (i, 0, 0)),
        ],
        out_specs=pl.BlockSpec((_BS, _H, _B), lambda i: (i, 0, 0)),
        out_shape=jax.ShapeDtypeStruct((_S, _H, _B), inputs.dtype),
        scratch_shapes=[pltpu.VMEM((1, _B), jnp.int32)],
    )(ids_t, emb3, x_t)
    return jnp.transpose(out_t, (2, 0, 1))
